# baseline (speedup 1.0000x reference)
import jax
import jax.numpy as jnp
from jax.experimental import pallas as pl
from jax.experimental.pallas import tpu as pltpu

_N = 10000
_NHID = 16
_NCLASS = 8
_BM = 200
_NB = _N // _BM
_G = 2000
_CW = 1664
_CWL = 1680
_NP2 = 20
_STEPS = _NB + _NP2


def _s1_body(x_ref, w1_ref, s1_ref):
    s1_ref[...] = jnp.dot(x_ref[...], w1_ref[...],
                          preferred_element_type=jnp.float32)


def _p2_gk(v):
    g = ((v >= 6).astype(jnp.int32) + (v >= 11) + (v >= 15) + (v >= 18))
    sub = jnp.where(g == 0, 0,
                    jnp.where(g == 1, 5,
                              jnp.where(g == 2, 9,
                                        jnp.where(g == 3, 12, 14))))
    return g, v - sub


def _main_body(b1_ref, w2_ref, b2_ref, s1_hbm, adj_ref, out_ref,
               p1buf, p2buf, s1_ref, s2_ref, s2m_ref,
               p1sem, p2sem, s1_sem):
    t = pl.program_id(0)

    def p1_copy(s):
        return pltpu.make_async_copy(
            adj_ref.at[pl.ds(s * _BM, _BM), :],
            p1buf.at[s % 2], p1sem.at[s % 2])

    def p2_copies(v):
        g, k = _p2_gk(v)
        sl = v % 2
        narrow = pltpu.make_async_copy(
            adj_ref.at[pl.ds(g * _G, _G), pl.ds(k * _CW, _CW)],
            p2buf.at[sl, :, pl.ds(0, _CW)], p2sem.at[sl])
        wide = pltpu.make_async_copy(
            adj_ref.at[pl.ds(g * _G, _G), pl.ds(_N - _CWL, _CWL)],
            p2buf.at[sl], p2sem.at[sl])
        return k, narrow, wide

    def issue(s):
        @pl.when(s < _NB)
        def _():
            p1_copy(s).start()

        @pl.when(s >= _NB)
        def _():
            k, narrow, wide = p2_copies(s - _NB)

            @pl.when(k < 5)
            def _():
                narrow.start()

            @pl.when(k == 5)
            def _():
                wide.start()

    @pl.when(t == 0)
    def _():
        pltpu.make_async_copy(s1_hbm, s1_ref, s1_sem).start()
        issue(0)
        s2m_ref[...] = jnp.zeros((_N, _NCLASS), jnp.float32)

    @pl.when(t + 1 < _STEPS)
    def _():
        issue(t + 1)

    @pl.when(t == 0)
    def _():
        pltpu.make_async_copy(s1_hbm, s1_ref, s1_sem).wait()

    @pl.when(t < _NB)
    def _():
        r = t
        p1_copy(r).wait()
        g = r // 10

        @pl.when((r > 0) & (r % 10 == 0))
        def _():
            s2m_ref[pl.ds((g - 1) * _CW, _CW), :] = (
                s2_ref[pl.ds((g - 1) * _CW, _CW), :])

        blk = p1buf[r % 2]
        h = jnp.dot(blk, s1_ref[...],
                    preferred_element_type=jnp.float32) + b1_ref[...]
        h = jnp.maximum(h, 0.0)
        s2_ref[pl.ds(r * _BM, _BM), :] = jnp.dot(
            h, w2_ref[...], preferred_element_type=jnp.float32)
        out_ref[pl.ds(r * _BM, _BM), :] = jnp.dot(
            blk, s2m_ref[...],
            preferred_element_type=jnp.float32) + b2_ref[...]

    @pl.when(t >= _NB)
    def _():
        v = t - _NB
        g, _k = _p2_gk(v)
        sl = v % 2
        k, narrow, wide = p2_copies(v)

        @pl.when(k < 5)
        def _():
            narrow.wait()
            out_ref[pl.ds(g * _G, _G), :] += jnp.dot(
                p2buf[sl, :, 0:_CW], s2_ref[pl.ds(k * _CW, _CW), :],
                preferred_element_type=jnp.float32)

        @pl.when(k == 5)
        def _():
            wide.wait()
            out_ref[pl.ds(g * _G, _G), :] += jnp.dot(
                p2buf[sl], s2_ref[pl.ds(_N - _CWL, _CWL), :],
                preferred_element_type=jnp.float32)


def kernel(x, adj, W1, b1, W2, b2):
    s1 = pl.pallas_call(
        _s1_body,
        out_shape=jax.ShapeDtypeStruct((_N, _NHID), jnp.float32),
    )(x, W1)

    b1r = b1.reshape(1, _NHID)
    b2r = b2.reshape(1, _NCLASS)

    return pl.pallas_call(
        _main_body,
        grid=(_STEPS,),
        in_specs=[
            pl.BlockSpec((1, _NHID), lambda t: (0, 0)),
            pl.BlockSpec((_NHID, _NCLASS), lambda t: (0, 0)),
            pl.BlockSpec((1, _NCLASS), lambda t: (0, 0)),
            pl.BlockSpec(memory_space=pltpu.MemorySpace.HBM),
            pl.BlockSpec(memory_space=pltpu.MemorySpace.HBM),
        ],
        out_specs=pl.BlockSpec((_N, _NCLASS), lambda t: (0, 0)),
        out_shape=jax.ShapeDtypeStruct((_N, _NCLASS), jnp.float32),
        scratch_shapes=[
            pltpu.VMEM((2, _BM, _N), jnp.float32),
            pltpu.VMEM((2, _G, _CWL), jnp.float32),
            pltpu.VMEM((_N, _NHID), jnp.float32),
            pltpu.VMEM((_N, _NCLASS), jnp.float32),
            pltpu.VMEM((_N, _NCLASS), jnp.float32),
            pltpu.SemaphoreType.DMA((2,)),
            pltpu.SemaphoreType.DMA((2,)),
            pltpu.SemaphoreType.DMA,
        ],
        compiler_params=pltpu.CompilerParams(
            vmem_limit_bytes=64 * 1024 * 1024,
        ),
    )(b1r, W2, b2r, s1, adj)


# speedup vs baseline: 1.0316x; 1.0316x over previous
import jax
import jax.numpy as jnp
from jax.experimental import pallas as pl
from jax.experimental.pallas import tpu as pltpu

_N = 10000
_NHID = 16
_NCLASS = 8
_BM = 400
_NB = _N // _BM
_GB = 5
_NG = _NB // _GB
_CW = 1664
_M = [0, 1664, 3328, 4992, 6656]
_STEPS = 2 * _NB


def _s1_body(x_ref, w1_ref, s1_ref):
    s1_ref[...] = jnp.dot(x_ref[...], w1_ref[...],
                          preferred_element_type=jnp.float32)


def _main_body(b1_ref, w2_ref, b2_ref, s1_hbm, adj_ref, out_ref,
               buf, s1_ref, s2_ref, s2m_ref, sem, s1_sem):
    t = pl.program_id(0)

    def p1_copy(s):
        return pltpu.make_async_copy(
            adj_ref.at[pl.ds(s * _BM, _BM), :],
            buf.at[s % 2], sem.at[s % 2])

    def p2_copy(v, gs):
        r0 = (v // _GB) * (_GB * _BM) + (v % _GB) * _BM
        m = _M[gs]
        return pltpu.make_async_copy(
            adj_ref.at[pl.ds(r0, _BM), pl.ds(m, _N - m)],
            buf.at[(v + _NB) % 2, :, pl.ds(m, _N - m)],
            sem.at[(v + _NB) % 2])

    def issue(s):
        @pl.when(s < _NB)
        def _():
            p1_copy(s).start()

        @pl.when(s >= _NB)
        def _():
            v = s - _NB
            g = v // _GB
            for gs in range(_NG):
                @pl.when(g == gs)
                def _(gs=gs):
                    p2_copy(v, gs).start()

    @pl.when(t == 0)
    def _():
        pltpu.make_async_copy(s1_hbm, s1_ref, s1_sem).start()
        issue(0)
        s2m_ref[...] = jnp.zeros((_N, _NCLASS), jnp.float32)

    @pl.when(t + 1 < _STEPS)
    def _():
        issue(t + 1)

    @pl.when(t == 0)
    def _():
        pltpu.make_async_copy(s1_hbm, s1_ref, s1_sem).wait()

    @pl.when(t < _NB)
    def _():
        r = t
        p1_copy(r).wait()
        g = r // _GB

        @pl.when((r > 0) & (r % _GB == 0))
        def _():
            s2m_ref[pl.ds((g - 1) * _CW, _CW), :] = (
                s2_ref[pl.ds((g - 1) * _CW, _CW), :])

        blk = buf[r % 2]
        h = jnp.dot(blk, s1_ref[...],
                    preferred_element_type=jnp.float32) + b1_ref[...]
        h = jnp.maximum(h, 0.0)
        s2_ref[pl.ds(r * _BM, _BM), :] = jnp.dot(
            h, w2_ref[...], preferred_element_type=jnp.float32)
        out_ref[pl.ds(r * _BM, _BM), :] = jnp.dot(
            blk, s2m_ref[...],
            preferred_element_type=jnp.float32) + b2_ref[...]

    @pl.when(t >= _NB)
    def _():
        v = t - _NB
        g = v // _GB
        r0 = g * (_GB * _BM) + (v % _GB) * _BM
        for gs in range(_NG):
            @pl.when(g == gs)
            def _(gs=gs):
                p2_copy(v, gs).wait()
                m = _M[gs]
                out_ref[pl.ds(r0, _BM), :] += jnp.dot(
                    buf[(v + _NB) % 2, :, m:_N],
                    s2_ref[pl.ds(m, _N - m), :],
                    preferred_element_type=jnp.float32)


def kernel(x, adj, W1, b1, W2, b2):
    s1 = pl.pallas_call(
        _s1_body,
        out_shape=jax.ShapeDtypeStruct((_N, _NHID), jnp.float32),
    )(x, W1)

    b1r = b1.reshape(1, _NHID)
    b2r = b2.reshape(1, _NCLASS)

    return pl.pallas_call(
        _main_body,
        grid=(_STEPS,),
        in_specs=[
            pl.BlockSpec((1, _NHID), lambda t: (0, 0)),
            pl.BlockSpec((_NHID, _NCLASS), lambda t: (0, 0)),
            pl.BlockSpec((1, _NCLASS), lambda t: (0, 0)),
            pl.BlockSpec(memory_space=pltpu.MemorySpace.HBM),
            pl.BlockSpec(memory_space=pltpu.MemorySpace.HBM),
        ],
        out_specs=pl.BlockSpec((_N, _NCLASS), lambda t: (0, 0)),
        out_shape=jax.ShapeDtypeStruct((_N, _NCLASS), jnp.float32),
        scratch_shapes=[
            pltpu.VMEM((2, _BM, _N), jnp.float32),
            pltpu.VMEM((_N, _NHID), jnp.float32),
            pltpu.VMEM((_N, _NCLASS), jnp.float32),
            pltpu.VMEM((_N, _NCLASS), jnp.float32),
            pltpu.SemaphoreType.DMA((2,)),
            pltpu.SemaphoreType.DMA,
        ],
        compiler_params=pltpu.CompilerParams(
            vmem_limit_bytes=64 * 1024 * 1024,
        ),
    )(b1r, W2, b2r, s1, adj)
